# baseline (speedup 1.0000x reference)
import functools

import jax
import jax.numpy as jnp
from jax import lax
from jax.experimental import pallas as pl
from jax.experimental.pallas import tpu as pltpu
from jax.experimental.pallas import tpu_sc as plsc

LOCAL = 12500
EMB = 128
ZERO_ROW = LOCAL
BATCH = 4096
SEQ = 200
TOKENS = BATCH * SEQ

NC, NS, L = 2, 16, 16
NW = NC * NS
TPW = TOKENS // NW
CHUNK = 128
NCHUNK = TPW // CHUNK


def _sc_body(ids_hbm, table_hbm, out_hbm, idx_v, rows_v, sem):
    wid = lax.axis_index("s") * NC + lax.axis_index("c")
    base = wid * TPW

    def chunk_body(c, carry):
        off = base + c * CHUNK
        pltpu.sync_copy(ids_hbm.at[pl.ds(off, CHUNK)], idx_v)

        def clamp_body(j, carry2):
            v = idx_v[pl.ds(j * L, L)]
            idx_v[pl.ds(j * L, L)] = jnp.minimum(v, ZERO_ROW)
            return carry2

        lax.fori_loop(0, CHUNK // L, clamp_body, 0)
        pltpu.async_copy(table_hbm.at[idx_v], rows_v, sem).wait()
        pltpu.sync_copy(rows_v, out_hbm.at[pl.ds(off, CHUNK)])
        return carry

    lax.fori_loop(0, NCHUNK, chunk_body, 0)


@functools.partial(
    pl.kernel,
    mesh=plsc.VectorSubcoreMesh(core_axis_name="c", subcore_axis_name="s"),
    out_type=jax.ShapeDtypeStruct((TOKENS, EMB), jnp.float32),
    scratch_types=[
        pltpu.VMEM((CHUNK,), jnp.int32),
        pltpu.VMEM((CHUNK, EMB), jnp.float32),
        pltpu.SemaphoreType.DMA,
    ],
)
def _sc_lookup(ids_hbm, table_hbm, out_hbm, idx_v, rows_v, sem):
    _sc_body(ids_hbm, table_hbm, out_hbm, idx_v, rows_v, sem)


def kernel(input_ids, weight):
    ids = input_ids.reshape(TOKENS)
    table = jnp.concatenate([weight, jnp.zeros((1, EMB), weight.dtype)], axis=0)
    out = _sc_lookup(ids, table)
    return out.reshape(BATCH, SEQ, EMB)


# speedup vs baseline: 1.0014x; 1.0014x over previous
import functools

import jax
import jax.numpy as jnp
from jax import lax
from jax.experimental import pallas as pl
from jax.experimental.pallas import tpu as pltpu
from jax.experimental.pallas import tpu_sc as plsc

LOCAL = 12500
EMB = 128
ZERO_ROW = LOCAL
BATCH = 4096
SEQ = 200
TOKENS = BATCH * SEQ

NC, NS, L = 2, 16, 16
NW = NC * NS
TPW = TOKENS // NW
CHUNK = 128
NCHUNK = TPW // CHUNK
NBUF = 4
DEPTH = 2


def _sc_body(ids_hbm, table_hbm, out_hbm, ids_v, rows_v, gsem, wsem):
    wid = lax.axis_index("s") * NC + lax.axis_index("c")
    base = wid * TPW

    pltpu.sync_copy(ids_hbm.at[pl.ds(base, TPW)], ids_v)

    def clamp_body(j, carry):
        v = ids_v[pl.ds(j * L, L)]
        ids_v[pl.ds(j * L, L)] = jnp.minimum(v, ZERO_ROW)
        return carry

    lax.fori_loop(0, TPW // L, clamp_body, 0)

    def issue_gather(c, b):
        pltpu.async_copy(
            table_hbm.at[ids_v.at[pl.ds(c * CHUNK, CHUNK)]], rows_v.at[b],
            gsem.at[b])

    def drain_gather(b):
        pltpu.make_async_copy(
            table_hbm.at[pl.ds(0, CHUNK)], rows_v.at[b], gsem.at[b]).wait()

    def issue_write(c, b):
        pltpu.async_copy(
            rows_v.at[b], out_hbm.at[pl.ds(base + c * CHUNK, CHUNK)],
            wsem.at[b])

    def drain_write(b):
        pltpu.make_async_copy(
            rows_v.at[b], out_hbm.at[pl.ds(0, CHUNK)], wsem.at[b]).wait()

    for d in range(DEPTH):
        issue_gather(d, d)

    def round_body(r, carry):
        for b in range(NBUF):
            c = r * NBUF + b
            bd = (b + DEPTH) % NBUF

            @pl.when(c + DEPTH < NCHUNK)
            def _():
                @pl.when(c + DEPTH >= NBUF)
                def _():
                    drain_write(bd)

                issue_gather(c + DEPTH, bd)

            drain_gather(b)
            issue_write(c, b)
        return carry

    lax.fori_loop(0, NCHUNK // NBUF, round_body, 0)

    for b in range(NBUF):
        drain_write(b)


@functools.partial(
    pl.kernel,
    mesh=plsc.VectorSubcoreMesh(core_axis_name="c", subcore_axis_name="s"),
    out_type=jax.ShapeDtypeStruct((TOKENS, EMB), jnp.float32),
    scratch_types=[
        pltpu.VMEM((TPW,), jnp.int32),
        pltpu.VMEM((NBUF, CHUNK, EMB), jnp.float32),
        pltpu.SemaphoreType.DMA((NBUF,)),
        pltpu.SemaphoreType.DMA((NBUF,)),
    ],
)
def _sc_lookup(ids_hbm, table_hbm, out_hbm, ids_v, rows_v, gsem, wsem):
    _sc_body(ids_hbm, table_hbm, out_hbm, ids_v, rows_v, gsem, wsem)


def kernel(input_ids, weight):
    ids = input_ids.reshape(TOKENS)
    table = jnp.concatenate([weight, jnp.zeros((1, EMB), weight.dtype)], axis=0)
    out = _sc_lookup(ids, table)
    return out.reshape(BATCH, SEQ, EMB)


# speedup vs baseline: 45.8474x; 45.7813x over previous
import functools

import jax
import jax.numpy as jnp
from jax import lax
from jax.experimental import pallas as pl
from jax.experimental.pallas import tpu as pltpu
from jax.experimental.pallas import tpu_sc as plsc

LOCAL = 12500
EMB = 128
NZ = 512
NROWS = LOCAL + NZ
BATCH = 4096
SEQ = 200
TOKENS = BATCH * SEQ

NC, NS, L = 2, 16, 16
NW = NC * NS
TPW = TOKENS // NW
CHUNK = 128
NCHUNK = TPW // CHUNK
NBUF = 4
DEPTH = 2


def _sc_body(ids_hbm, table_hbm, out_hbm, ids_v, rows_v, gsem, wsem):
    cid = lax.axis_index("c")
    sid = lax.axis_index("s")
    wid = sid * NC + cid
    base = wid * TPW

    del cid

    pltpu.sync_copy(ids_hbm.at[pl.ds(base, TPW)], ids_v)

    def clamp_body(j, carry):
        v = ids_v[pl.ds(j * L, L)]
        zrow = LOCAL + (v & (NZ - 1))
        ids_v[pl.ds(j * L, L)] = jnp.where(v < LOCAL, v, zrow)
        return carry

    lax.fori_loop(0, TPW // L, clamp_body, 0)
    def issue_gather(c, b):
        pltpu.async_copy(
            table_hbm.at[ids_v.at[pl.ds(c * CHUNK, CHUNK)]], rows_v.at[b],
            gsem.at[b])

    def drain_gather(b):
        pltpu.make_async_copy(
            table_hbm.at[pl.ds(0, CHUNK)], rows_v.at[b], gsem.at[b]).wait()

    def issue_write(c, b):
        pltpu.async_copy(
            rows_v.at[b], out_hbm.at[pl.ds(base + c * CHUNK, CHUNK)],
            wsem.at[b])

    def drain_write(b):
        pltpu.make_async_copy(
            rows_v.at[b], out_hbm.at[pl.ds(0, CHUNK)], wsem.at[b]).wait()

    for d in range(DEPTH):
        issue_gather(d, d)

    def round_body(r, carry):
        for b in range(NBUF):
            c = r * NBUF + b
            bd = (b + DEPTH) % NBUF

            @pl.when(c + DEPTH < NCHUNK)
            def _():
                @pl.when(c + DEPTH >= NBUF)
                def _():
                    drain_write(bd)

                issue_gather(c + DEPTH, bd)

            drain_gather(b)
            issue_write(c, b)
        return carry

    lax.fori_loop(0, NCHUNK // NBUF, round_body, 0)

    for b in range(NBUF):
        drain_write(b)


@functools.partial(
    pl.kernel,
    mesh=plsc.VectorSubcoreMesh(core_axis_name="c", subcore_axis_name="s"),
    out_type=jax.ShapeDtypeStruct((TOKENS, EMB), jnp.float32),
    scratch_types=[
        pltpu.VMEM((TPW,), jnp.int32),
        pltpu.VMEM((NBUF, CHUNK, EMB), jnp.float32),
        pltpu.SemaphoreType.DMA((NBUF,)),
        pltpu.SemaphoreType.DMA((NBUF,)),
    ],
)
def _sc_lookup(ids_hbm, table_hbm, out_hbm, ids_v, rows_v, gsem, wsem):
    _sc_body(ids_hbm, table_hbm, out_hbm, ids_v, rows_v, gsem, wsem)


def kernel(input_ids, weight):
    ids = input_ids.reshape(TOKENS)
    table = jnp.concatenate([weight, jnp.zeros((NZ, EMB), weight.dtype)],
                            axis=0)
    out = _sc_lookup(ids, table)
    return out.reshape(BATCH, SEQ, EMB)


# speedup vs baseline: 79.1726x; 1.7269x over previous
import functools

import jax
import jax.numpy as jnp
from jax import lax
from jax.experimental import pallas as pl
from jax.experimental.pallas import tpu as pltpu
from jax.experimental.pallas import tpu_sc as plsc

LOCAL = 12500
EMB = 128
NZ = 4096
NROWS = LOCAL + NZ
BATCH = 4096
SEQ = 200
TOKENS = BATCH * SEQ

NC, NS, L = 2, 16, 16
NW = NC * NS
TPW = TOKENS // NW
CHUNK = 128
NCHUNK = TPW // CHUNK
NBUF = 5
DEPTH = 3


def _sc_body(ids_hbm, table_hbm, out_hbm, ids_v, rows_v, gsem, wsem):
    cid = lax.axis_index("c")
    sid = lax.axis_index("s")
    wid = sid * NC + cid
    base = wid * TPW

    del cid

    pltpu.sync_copy(ids_hbm.at[pl.ds(base, TPW)], ids_v)

    def clamp_body(j, carry):
        v = ids_v[pl.ds(j * L, L)]
        zrow = LOCAL + (v & (NZ - 1))
        ids_v[pl.ds(j * L, L)] = jnp.where(v < LOCAL, v, zrow)
        return carry

    lax.fori_loop(0, TPW // L, clamp_body, 0)
    def issue_gather(c, b):
        pltpu.async_copy(
            table_hbm.at[ids_v.at[pl.ds(c * CHUNK, CHUNK)]], rows_v.at[b],
            gsem.at[b])

    def drain_gather(b):
        pltpu.make_async_copy(
            table_hbm.at[pl.ds(0, CHUNK)], rows_v.at[b], gsem.at[b]).wait()

    def issue_write(c, b):
        pltpu.async_copy(
            rows_v.at[b], out_hbm.at[pl.ds(base + c * CHUNK, CHUNK)],
            wsem.at[b])

    def drain_write(b):
        pltpu.make_async_copy(
            rows_v.at[b], out_hbm.at[pl.ds(0, CHUNK)], wsem.at[b]).wait()

    for d in range(DEPTH):
        issue_gather(d, d)

    def round_body(r, carry):
        for b in range(NBUF):
            c = r * NBUF + b
            bd = (b + DEPTH) % NBUF

            @pl.when(c + DEPTH < NCHUNK)
            def _():
                @pl.when(c + DEPTH >= NBUF)
                def _():
                    drain_write(bd)

                issue_gather(c + DEPTH, bd)

            drain_gather(b)
            issue_write(c, b)
        return carry

    lax.fori_loop(0, NCHUNK // NBUF, round_body, 0)

    for b in range(NBUF):
        drain_write(b)


@functools.partial(
    pl.kernel,
    mesh=plsc.VectorSubcoreMesh(core_axis_name="c", subcore_axis_name="s"),
    out_type=jax.ShapeDtypeStruct((TOKENS, EMB), jnp.float32),
    scratch_types=[
        pltpu.VMEM((TPW,), jnp.int32),
        pltpu.VMEM((NBUF, CHUNK, EMB), jnp.float32),
        pltpu.SemaphoreType.DMA((NBUF,)),
        pltpu.SemaphoreType.DMA((NBUF,)),
    ],
)
def _sc_lookup(ids_hbm, table_hbm, out_hbm, ids_v, rows_v, gsem, wsem):
    _sc_body(ids_hbm, table_hbm, out_hbm, ids_v, rows_v, gsem, wsem)


def kernel(input_ids, weight):
    ids = input_ids.reshape(TOKENS)
    table = jnp.concatenate([weight, jnp.zeros((NZ, EMB), weight.dtype)],
                            axis=0)
    out = _sc_lookup(ids, table)
    return out.reshape(BATCH, SEQ, EMB)


# speedup vs baseline: 133.3537x; 1.6843x over previous
import functools

import jax
import jax.numpy as jnp
from jax import lax
from jax.experimental import pallas as pl
from jax.experimental.pallas import tpu as pltpu
from jax.experimental.pallas import tpu_sc as plsc

LOCAL = 12500
EMB = 128
NZ = 64
NROWS = LOCAL + NZ
BATCH = 4096
SEQ = 200
TOKENS = BATCH * SEQ

NC, NS, L = 2, 16, 16
NW = NC * NS
TPW = TOKENS // NW
CHUNK = 64
IBLK = 1024
NBLK = TPW // IBLK
CPB = IBLK // CHUNK
NCHUNK = TPW // CHUNK


def _sc_body(ids_hbm, table_hbm, out_hbm, ids_v, rows_v, table_sp, gsem,
             wsem, isem):
    sid = lax.axis_index("s")
    wid = sid * NC + lax.axis_index("c")
    base = wid * TPW

    @pl.when(sid == 0)
    def _():
        pltpu.sync_copy(table_hbm, table_sp)

    def drain_ids(slot):
        pltpu.make_async_copy(
            ids_hbm.at[pl.ds(0, IBLK)], ids_v.at[pl.ds(0, IBLK)],
            isem.at[slot]).wait()

    def issue_gather(islot, cc, b):
        pltpu.async_copy(
            table_sp.at[ids_v.at[pl.ds(islot * IBLK + cc * CHUNK, CHUNK)]],
            rows_v.at[pl.ds(b * CHUNK, CHUNK)], gsem.at[b])

    def drain_gather(b):
        pltpu.make_async_copy(
            table_sp.at[pl.ds(0, CHUNK)], rows_v.at[pl.ds(0, CHUNK)],
            gsem.at[b]).wait()

    def issue_write(c, b):
        pltpu.async_copy(
            rows_v.at[pl.ds(b * CHUNK, CHUNK)],
            out_hbm.at[pl.ds(base + c * CHUNK, CHUNK)], wsem.at[b])

    def drain_write(b):
        pltpu.make_async_copy(
            rows_v.at[pl.ds(0, CHUNK)], out_hbm.at[pl.ds(0, CHUNK)],
            wsem.at[b]).wait()

    for blk0 in range(2):
        pltpu.async_copy(
            ids_hbm.at[pl.ds(base + blk0 * IBLK, IBLK)],
            ids_v.at[pl.ds(blk0 * IBLK, IBLK)], isem.at[blk0])
    plsc.subcore_barrier()

    def clamp(islot, cc):
        for j in range(CHUNK // L):
            o = islot * IBLK + cc * CHUNK + j * L
            v = ids_v[pl.ds(o, L)]
            zrow = LOCAL + (v & (NZ - 1))
            ids_v[pl.ds(o, L)] = jnp.where(v < LOCAL, v, zrow)

    def block_loop(blk, carry):
        for slot in range(2):
            @pl.when(blk % 2 == slot)
            def _():
                drain_ids(slot)

                for cc in range(CPB):
                    b = cc % 2
                    c = blk * CPB + cc

                    @pl.when(c >= 2)
                    def _():
                        drain_write(b)

                    clamp(slot, cc)
                    issue_gather(slot, cc, b)

                    @pl.when(c >= 1)
                    def _():
                        drain_gather(1 - b)
                        issue_write(c - 1, 1 - b)

                @pl.when(blk + 2 < NBLK)
                def _():
                    pltpu.async_copy(
                        ids_hbm.at[pl.ds(base + (blk + 2) * IBLK, IBLK)],
                        ids_v.at[pl.ds(slot * IBLK, IBLK)], isem.at[slot])
        return carry

    lax.fori_loop(0, NBLK, block_loop, 0)

    last = NCHUNK - 1
    bl = last % 2
    drain_gather(bl)
    issue_write(last, bl)
    drain_write(1 - bl)
    drain_write(bl)


@functools.partial(
    pl.kernel,
    mesh=plsc.VectorSubcoreMesh(core_axis_name="c", subcore_axis_name="s"),
    out_type=jax.ShapeDtypeStruct((TOKENS, EMB), jnp.float32),
    scratch_types=[
        pltpu.VMEM((2 * IBLK,), jnp.int32),
        pltpu.VMEM((2 * CHUNK, EMB), jnp.float32),
        pltpu.VMEM_SHARED((NROWS, EMB), jnp.float32),
        pltpu.SemaphoreType.DMA((2,)),
        pltpu.SemaphoreType.DMA((2,)),
        pltpu.SemaphoreType.DMA((2,)),
    ],
)
def _sc_lookup(ids_hbm, table_hbm, out_hbm, ids_v, rows_v, table_sp, gsem,
               wsem, isem):
    _sc_body(ids_hbm, table_hbm, out_hbm, ids_v, rows_v, table_sp, gsem,
             wsem, isem)


def kernel(input_ids, weight):
    ids = input_ids.reshape(TOKENS)
    table = jnp.concatenate([weight, jnp.zeros((NZ, EMB), weight.dtype)],
                            axis=0)
    out = _sc_lookup(ids, table)
    return out.reshape(BATCH, SEQ, EMB)
